# Initial kernel scaffold; baseline (speedup 1.0000x reference)
import jax
import jax.numpy as jnp
from jax.experimental import pallas as pl


def kernel(outputs_support, outputs_delete, targets, masks):
    raise NotImplementedError("write your pallas kernel here")


# speedup vs baseline: 7.5737x; 7.5737x over previous
import functools

import jax
import jax.numpy as jnp
from jax import lax
from jax.experimental import pallas as pl
from jax.experimental.pallas import tpu as pltpu
from jax.experimental.pallas import tpu_sc as plsc

ROWS = 128
COLS = 32768
Q_IDX = int(COLS * (1 - 0.1))
TARGET = COLS - Q_IDX
NB1 = 2048
NB2 = 1024
NB3 = 1024
L = 16


def _zero(ref, nb):
    zv = jnp.zeros((L,), ref.dtype)

    def body(j, c):
        ref[pl.ds(j * L, L)] = zv
        return c

    lax.fori_loop(0, nb // L, body, jnp.int32(0))


def _walk(cnt_ref, sum_ref, nb, t):
    iota = lax.iota(jnp.int32, L)

    def body(k, carry):
        found, b_sel, ab_cnt, ab_sum, acc_cnt, acc_sum = carry
        base = nb - L * (k + 1)
        cv = cnt_ref[pl.ds(base, L)]
        sv = sum_ref[pl.ds(base, L)]
        rc = lax.rev(cv, (0,))
        rs = lax.rev(sv, (0,))
        c = lax.cumsum(rc, axis=0)
        blk_cnt = jnp.max(c)
        blk_sum = jnp.sum(rs)
        hit = jnp.where((found == 0) & (acc_cnt + blk_cnt >= t),
                        jnp.int32(1), jnp.int32(0))
        ge = (acc_cnt + c) >= t
        i_star = jnp.min(jnp.where(ge, iota, jnp.int32(L)))
        within_cnt = jnp.sum(jnp.where(iota < i_star, rc, jnp.int32(0)))
        within_sum = jnp.sum(jnp.where(iota < i_star, rs, jnp.float32(0.0)))
        b_new = base + jnp.int32(L - 1) - i_star
        b_sel = jnp.where(hit == 1, b_new, b_sel)
        ab_cnt = jnp.where(hit == 1, acc_cnt + within_cnt, ab_cnt)
        ab_sum = jnp.where(hit == 1, acc_sum + within_sum, ab_sum)
        found = found | hit
        acc_cnt = jnp.where(found == 1, acc_cnt, acc_cnt + blk_cnt)
        acc_sum = jnp.where(found == 1, acc_sum, acc_sum + blk_sum)
        return (found, b_sel, ab_cnt, ab_sum, acc_cnt, acc_sum)

    init = (jnp.int32(0), jnp.int32(0), jnp.int32(0), jnp.float32(0.0),
            jnp.int32(0), jnp.float32(0.0))
    found, b_sel, ab_cnt, ab_sum, _, _ = lax.fori_loop(0, nb // L, body, init)
    return b_sel, ab_cnt, ab_sum


def _pass1(row_ref, cnt_ref, sum_ref):
    ones = jnp.full((L,), 1, jnp.int32)

    def body(j, c):
        v = row_ref[pl.ds(j * L, L)]
        a = jnp.abs(v)
        bits = lax.bitcast_convert_type(a, jnp.int32)
        k1 = lax.shift_right_logical(bits, 20)
        plsc.addupdate_scatter(cnt_ref, [k1], ones)
        plsc.addupdate_scatter(sum_ref, [k1], a)
        return c

    lax.fori_loop(0, COLS // L, body, jnp.int32(0))


def _pass_masked(row_ref, cnt_ref, sum_ref, pre_shift, prefix, key_shift,
                 key_mask):
    ones = jnp.full((L,), 1, jnp.int32)

    def body(j, c):
        v = row_ref[pl.ds(j * L, L)]
        a = jnp.abs(v)
        bits = lax.bitcast_convert_type(a, jnp.int32)
        m = lax.shift_right_logical(bits, pre_shift) == prefix
        key = lax.shift_right_logical(bits, key_shift) & key_mask
        plsc.addupdate_scatter(cnt_ref, [key], ones, mask=m)
        plsc.addupdate_scatter(sum_ref, [key], a, mask=m)
        return c

    lax.fori_loop(0, COLS // L, body, jnp.int32(0))


def _make_selector():
    info = plsc.get_sparse_core_info()
    nw = info.num_cores * info.num_subcores
    rows_per_w = ROWS // nw
    mesh = plsc.VectorSubcoreMesh(core_axis_name="c", subcore_axis_name="s")

    @functools.partial(
        pl.kernel,
        mesh=mesh,
        compiler_params=pltpu.CompilerParams(needs_layout_passes=False),
        out_type=jax.ShapeDtypeStruct((nw, L), jnp.float32),
        scratch_types=[
            pltpu.VMEM((COLS,), jnp.float32),
            pltpu.VMEM((NB1,), jnp.int32),
            pltpu.VMEM((NB1,), jnp.float32),
            pltpu.VMEM((NB2,), jnp.int32),
            pltpu.VMEM((NB2,), jnp.float32),
            pltpu.VMEM((L,), jnp.float32),
        ],
    )
    def sel(masks_hbm, out_hbm, row_v, cnt1, sum1, cnt2, sum2, stage):
        wid = lax.axis_index("s") * info.num_cores + lax.axis_index("c")
        iota = lax.iota(jnp.int32, L)

        def row_body(r, acc_vec):
            row = wid * rows_per_w + r
            pltpu.sync_copy(masks_hbm.at[row], row_v)
            _zero(cnt1, NB1)
            _zero(sum1, NB1)
            _pass1(row_v, cnt1, sum1)
            b1, ac1, as1 = _walk(cnt1, sum1, NB1, jnp.int32(TARGET))
            t2 = jnp.int32(TARGET) - ac1
            _zero(cnt2, NB2)
            _zero(sum2, NB2)
            _pass_masked(row_v, cnt2, sum2, 20, b1, 10, jnp.int32(NB2 - 1))
            b2, ac2, as2 = _walk(cnt2, sum2, NB2, t2)
            t3 = t2 - ac2
            _zero(cnt2, NB2)
            _zero(sum2, NB2)
            pre3 = lax.shift_left(b1, 10) | b2
            _pass_masked(row_v, cnt2, sum2, 10, pre3, 0, jnp.int32(NB3 - 1))
            _, _, as3 = _walk(cnt2, sum2, NB3, t3)
            ans = as1 + as2 + as3
            return jnp.where(iota == r, ans, acc_vec)

        acc = lax.fori_loop(0, rows_per_w, row_body,
                            jnp.zeros((L,), jnp.float32))
        stage[...] = acc
        pltpu.sync_copy(stage, out_hbm.at[wid])

    return sel


_selector = _make_selector()


def kernel(outputs_support, outputs_delete, targets, masks):
    parts = _selector(masks)
    return outputs_support[0] + 0.01 * jnp.sum(parts)


# speedup vs baseline: 7.6336x; 1.0079x over previous
import functools

import jax
import jax.numpy as jnp
from jax import lax
from jax.experimental import pallas as pl
from jax.experimental.pallas import tpu as pltpu
from jax.experimental.pallas import tpu_sc as plsc

ROWS = 128
COLS = 32768
Q_IDX = int(COLS * (1 - 0.1))
TARGET = COLS - Q_IDX
NB1 = 2048
NB2 = 1024
L = 16


def _zero2(cnt_ref, sum_ref, nb):
    zi = jnp.zeros((L,), jnp.int32)
    zf = jnp.zeros((L,), jnp.float32)

    def body(j, c):
        cnt_ref[pl.ds(j * L, L)] = zi
        sum_ref[pl.ds(j * L, L)] = zf
        return c

    lax.fori_loop(0, nb // L, body, jnp.int32(0))


def _walk(cnt_ref, sum_ref, start_blk, t):
    iota = lax.iota(jnp.int32, L)

    def cond(carry):
        return (carry[1] == 0) & (carry[0] >= 0)

    def body(carry):
        k, found, b_sel, ab_cnt, ab_sum, acc_cnt, acc_sum = carry
        base = k * L
        cv = cnt_ref[pl.ds(base, L)]
        sv = sum_ref[pl.ds(base, L)]
        rc = lax.rev(cv, (0,))
        rs = lax.rev(sv, (0,))
        c = lax.cumsum(rc, axis=0)
        blk_cnt = jnp.max(c)
        blk_sum = jnp.sum(rs)
        hit = jnp.where(acc_cnt + blk_cnt >= t, jnp.int32(1), jnp.int32(0))
        ge = (acc_cnt + c) >= t
        i_star = jnp.min(jnp.where(ge, iota, jnp.int32(L)))
        within_cnt = jnp.sum(jnp.where(iota < i_star, rc, jnp.int32(0)))
        within_sum = jnp.sum(jnp.where(iota < i_star, rs, jnp.float32(0.0)))
        b_new = base + jnp.int32(L - 1) - i_star
        b_sel = jnp.where(hit == 1, b_new, b_sel)
        ab_cnt = jnp.where(hit == 1, acc_cnt + within_cnt, ab_cnt)
        ab_sum = jnp.where(hit == 1, acc_sum + within_sum, ab_sum)
        acc_cnt = jnp.where(hit == 1, acc_cnt, acc_cnt + blk_cnt)
        acc_sum = jnp.where(hit == 1, acc_sum, acc_sum + blk_sum)
        return (k - jnp.int32(1), found | hit, b_sel, ab_cnt, ab_sum,
                acc_cnt, acc_sum)

    init = (start_blk, jnp.int32(0), jnp.int32(0), jnp.int32(0),
            jnp.float32(0.0), jnp.int32(0), jnp.float32(0.0))
    out = lax.while_loop(cond, body, init)
    return out[2], out[3], out[4]


def _pass1(row_ref, cnt_ref, sum_ref):
    ones = jnp.full((L,), 1, jnp.int32)
    UN = 8

    def body(j, kmax):
        for u in range(UN):
            v = row_ref[pl.ds((j * UN + u) * L, L)]
            a = jnp.abs(v)
            bits = lax.bitcast_convert_type(a, jnp.int32)
            k1 = lax.shift_right_logical(bits, 20)
            plsc.addupdate_scatter(cnt_ref, [k1], ones)
            plsc.addupdate_scatter(sum_ref, [k1], a)
            kmax = jnp.maximum(kmax, k1)
        return kmax

    kmax = lax.fori_loop(0, COLS // L // UN, body, jnp.zeros((L,), jnp.int32))
    return jnp.max(kmax)


def _pass23(row_ref, cnt_ref, sum_ref, pre_shift, prefix, key_shift):
    ones = jnp.full((L,), 1, jnp.int32)
    km = jnp.int32(NB2 - 1)
    UN = 4

    def body(j, kmax):
        for u in range(UN):
            v = row_ref[pl.ds((j * UN + u) * L, L)]
            a = jnp.abs(v)
            bits = lax.bitcast_convert_type(a, jnp.int32)
            m = lax.shift_right_logical(bits, pre_shift) == prefix
            key = lax.shift_right_logical(bits, key_shift) & km
            plsc.addupdate_scatter(cnt_ref, [key], ones, mask=m)
            plsc.addupdate_scatter(sum_ref, [key], a, mask=m)
            kmax = jnp.maximum(kmax, jnp.where(m, key, jnp.int32(0)))
        return kmax

    kmax = lax.fori_loop(0, COLS // L // UN, body, jnp.zeros((L,), jnp.int32))
    return jnp.max(kmax)


def _select_row(row_ref, cnt1, sum1, cnt2, sum2):
    _zero2(cnt1, sum1, NB1)
    kmax1 = _pass1(row_ref, cnt1, sum1)
    b1, ac1, as1 = _walk(cnt1, sum1, lax.shift_right_logical(kmax1, 4),
                         jnp.int32(TARGET))
    t2 = jnp.int32(TARGET) - ac1
    _zero2(cnt2, sum2, NB2)
    kmax2 = _pass23(row_ref, cnt2, sum2, 20, b1, 10)
    b2, ac2, as2 = _walk(cnt2, sum2, lax.shift_right_logical(kmax2, 4), t2)
    t3 = t2 - ac2
    _zero2(cnt2, sum2, NB2)
    pre3 = lax.shift_left(b1, 10) | b2
    kmax3 = _pass23(row_ref, cnt2, sum2, 10, pre3, 0)
    _, _, as3 = _walk(cnt2, sum2, lax.shift_right_logical(kmax3, 4), t3)
    return as1 + as2 + as3


def _make_selector():
    info = plsc.get_sparse_core_info()
    nw = info.num_cores * info.num_subcores
    rows_per_w = ROWS // nw
    mesh = plsc.VectorSubcoreMesh(core_axis_name="c", subcore_axis_name="s")

    @functools.partial(
        pl.kernel,
        mesh=mesh,
        compiler_params=pltpu.CompilerParams(needs_layout_passes=False),
        out_type=jax.ShapeDtypeStruct((nw, L), jnp.float32),
        scratch_types=[
            pltpu.VMEM((COLS,), jnp.float32),
            pltpu.VMEM((COLS,), jnp.float32),
            pltpu.VMEM((NB1,), jnp.int32),
            pltpu.VMEM((NB1,), jnp.float32),
            pltpu.VMEM((NB2,), jnp.int32),
            pltpu.VMEM((NB2,), jnp.float32),
            pltpu.VMEM((L,), jnp.float32),
            pltpu.SemaphoreType.DMA,
            pltpu.SemaphoreType.DMA,
        ],
    )
    def sel(masks_hbm, out_hbm, row_a, row_b, cnt1, sum1, cnt2, sum2, stage,
            sem_a, sem_b):
        wid = lax.axis_index("s") * info.num_cores + lax.axis_index("c")
        row0 = wid * rows_per_w
        iota = lax.iota(jnp.int32, L)
        bufs = (row_a, row_b)
        sems = (sem_a, sem_b)

        handles = {}
        for r in range(2):
            handles[r] = pltpu.async_copy(masks_hbm.at[row0 + r], bufs[r],
                                          sems[r])
        acc = jnp.zeros((L,), jnp.float32)
        for r in range(rows_per_w):
            handles[r].wait()
            ans = _select_row(bufs[r % 2], cnt1, sum1, cnt2, sum2)
            acc = jnp.where(iota == r, ans, acc)
            if r + 2 < rows_per_w:
                handles[r + 2] = pltpu.async_copy(
                    masks_hbm.at[row0 + r + 2], bufs[r % 2], sems[r % 2])
        stage[...] = acc
        pltpu.sync_copy(stage, out_hbm.at[wid])

    return sel


_selector = _make_selector()


def kernel(outputs_support, outputs_delete, targets, masks):
    parts = _selector(masks)
    return outputs_support[0] + 0.01 * jnp.sum(parts)
